# baseline (speedup 1.0000x reference)
import jax
import jax.numpy as jnp
from jax.experimental import pallas as pl
from jax.experimental.pallas import tpu as pltpu

N = 512
D = 128
BI = 32


def _mp_kernel(x_ref, adj_ref, ph_ref, w1r_ref, w1s_ref, b1_ref, w2_ref,
               b2_ref, wg2_ref, u1x_ref, u1a_ref, c1_ref, u2_ref,
               c2_ref, out_ref, hs_ref, cs_ref):
    i = pl.program_id(0)

    @pl.when(i == 0)
    def _prep():
        hs_ref[...] = jnp.dot(x_ref[...], w1s_ref[...],
                              preferred_element_type=jnp.float32)
        ph = ph_ref[...]
        cs_ref[...] = jnp.concatenate(
            [jnp.cos(ph), jnp.sin(ph), jnp.ones_like(ph[:, :1])], axis=-1)

    xb = x_ref[pl.ds(i * BI, BI), :]

    hr = jnp.dot(xb, w1r_ref[...],
                 preferred_element_type=jnp.float32) + b1_ref[...]
    hs = hs_ref[...]

    cs = cs_ref[...]
    csb = cs_ref[pl.ds(i * BI, BI), :]
    cd = (csb[:, None, :] * cs[None, :, :]).reshape(BI * N, cs.shape[-1])
    t = jnp.tanh(jnp.dot(cd, wg2_ref[...],
                         preferred_element_type=jnp.float32))

    hid = jax.nn.relu((hr[:, None, :] + hs[None, :, :]).reshape(BI * N, D))
    msg = jnp.dot(hid, w2_ref[...],
                  preferred_element_type=jnp.float32)
    mt = msg * t

    m = (adj_ref[...] != 0).astype(jnp.float32)
    counts = jnp.sum(m, axis=1, keepdims=True)
    mw = m * (0.5 / jnp.maximum(counts, 1.0))
    rs = jnp.sum(mw, axis=1, keepdims=True)
    bdot = lambda w, v: jax.lax.dot_general(
        w, v.reshape(BI, N, D),
        dimension_numbers=(((1,), (1,)), ((0,), (0,))),
        preferred_element_type=jnp.float32)
    agg = bdot(mw, msg) + bdot(mw, mt) + b2_ref[...] * (rs + bdot(mw, t))

    h = jax.nn.relu(
        jnp.dot(xb, u1x_ref[...], preferred_element_type=jnp.float32)
        + jnp.dot(agg, u1a_ref[...], preferred_element_type=jnp.float32)
        + c1_ref[...])
    out_ref[...] = xb + jnp.dot(h, u2_ref[...],
                                preferred_element_type=jnp.float32) + c2_ref[...]


@jax.jit
def kernel(node_features, adjacency, node_phases, W1, b1, W2, b2, Wg, bg,
           U1, c1, U2, c2):
    d = node_features.shape[1]
    osc = node_phases.shape[1]
    full = lambda shape: pl.BlockSpec(shape, lambda i: (0,) * len(shape))
    grid = N // BI
    wg2 = jnp.concatenate([Wg, Wg, bg[None, :]], axis=0) * 0.5
    return pl.pallas_call(
        _mp_kernel,
        grid=(grid,),
        in_specs=[
            full((N, D)),
            pl.BlockSpec((BI, N), lambda i: (i, 0)),
            full(node_phases.shape),
            full((D, D)), full((D, D)), full((D,)),
            full((D, D)), full((D,)),
            full((2 * osc + 1, D)),
            full((D, D)), full((D, D)), full((D,)),
            full((D, D)), full((D,)),
        ],
        out_specs=pl.BlockSpec((BI, D), lambda i: (i, 0)),
        out_shape=jax.ShapeDtypeStruct((N, D), jnp.float32),
        scratch_shapes=[
            pltpu.VMEM((N, D), jnp.float32),
            pltpu.VMEM((N, 2 * osc + 1), jnp.float32),
        ],
    )(node_features, adjacency, node_phases,
      W1[:d], W1[d:], b1, W2, b2, wg2, U1[:d], U1[d:], c1, U2, c2)


# speedup vs baseline: 1.4431x; 1.4431x over previous
import jax
import jax.numpy as jnp
from jax.experimental import pallas as pl
from jax.experimental.pallas import tpu as pltpu

N = 512
D = 128
BI = 32


def _mp_kernel(x_ref, adj_ref, ph_ref, w1r_ref, w1s_ref, b1_ref, w2_ref,
               b2_ref, wg2_ref, u1x_ref, u1a_ref, c1_ref, u2_ref,
               c2_ref, out_ref, hs_ref, cs_ref):
    i = pl.program_id(0)

    @pl.when(i == 0)
    def _prep():
        hs_ref[...] = jnp.dot(x_ref[...], w1s_ref[...],
                              preferred_element_type=jnp.float32)
        ph = ph_ref[...]
        cs_ref[...] = jnp.concatenate(
            [jnp.cos(ph), jnp.sin(ph), jnp.ones_like(ph[:, :1])], axis=-1)

    xb = x_ref[pl.ds(i * BI, BI), :]

    hr = jnp.dot(xb, w1r_ref[...],
                 preferred_element_type=jnp.float32) + b1_ref[...]
    hs = hs_ref[...]

    cs = cs_ref[...]
    csb = cs_ref[pl.ds(i * BI, BI), :]
    cd = (csb[:, None, :] * cs[None, :, :]).reshape(BI * N, cs.shape[-1])
    t = jnp.tanh(jnp.dot(cd, wg2_ref[...],
                         preferred_element_type=jnp.float32))

    hid = jax.nn.relu((hr[:, None, :] + hs[None, :, :]).reshape(BI * N, D))
    mb = jnp.dot(hid, w2_ref[...],
                 preferred_element_type=jnp.float32) + b2_ref[...]
    prod = mb * t + mb

    m = (adj_ref[...] != 0).astype(jnp.float32)
    counts = jnp.sum(m, axis=1, keepdims=True)
    mw = m * (0.5 / jnp.maximum(counts, 1.0))
    agg = jax.lax.dot_general(
        mw, prod.reshape(BI, N, D),
        dimension_numbers=(((1,), (1,)), ((0,), (0,))),
        preferred_element_type=jnp.float32)

    h = jax.nn.relu(
        jnp.dot(xb, u1x_ref[...], preferred_element_type=jnp.float32)
        + jnp.dot(agg, u1a_ref[...], preferred_element_type=jnp.float32)
        + c1_ref[...])
    out_ref[...] = xb + jnp.dot(h, u2_ref[...],
                                preferred_element_type=jnp.float32) + c2_ref[...]


@jax.jit
def kernel(node_features, adjacency, node_phases, W1, b1, W2, b2, Wg, bg,
           U1, c1, U2, c2):
    d = node_features.shape[1]
    osc = node_phases.shape[1]
    full = lambda shape: pl.BlockSpec(shape, lambda i: (0,) * len(shape))
    grid = N // BI
    wg2 = jnp.concatenate([Wg, Wg, bg[None, :]], axis=0) * 0.5
    return pl.pallas_call(
        _mp_kernel,
        grid=(grid,),
        in_specs=[
            full((N, D)),
            pl.BlockSpec((BI, N), lambda i: (i, 0)),
            full(node_phases.shape),
            full((D, D)), full((D, D)), full((D,)),
            full((D, D)), full((D,)),
            full((2 * osc + 1, D)),
            full((D, D)), full((D, D)), full((D,)),
            full((D, D)), full((D,)),
        ],
        out_specs=pl.BlockSpec((BI, D), lambda i: (i, 0)),
        out_shape=jax.ShapeDtypeStruct((N, D), jnp.float32),
        scratch_shapes=[
            pltpu.VMEM((N, D), jnp.float32),
            pltpu.VMEM((N, 2 * osc + 1), jnp.float32),
        ],
    )(node_features, adjacency, node_phases,
      W1[:d], W1[d:], b1, W2, b2, wg2, U1[:d], U1[d:], c1, U2, c2)
